# baseline (speedup 1.0000x reference)
import functools

import jax
import jax.numpy as jnp
from jax import lax
from jax.experimental import pallas as pl
from jax.experimental.pallas import tpu as pltpu
from jax.experimental.pallas import tpu_sc as plsc

_N = 10000
_E = 320000
_DF = 128
_DE = 16
_H = 256
_HH = 128
_B = 256
_TASKS = 12
_DEPTH = 3

_EPAD = 327680
_IDX_ROWS = _EPAD // 128
_ACC = 10240
_NSUB = 16
_ROWS_T = _IDX_ROWS // _NSUB
_ACC_T = _ACC // _NSUB
_SLAB = 32

_ECH = _E // 128
_ECH_T = 156


@functools.cache
def _sc_fns():
    mesh = plsc.VectorSubcoreMesh(core_axis_name="c", subcore_axis_name="s")

    @functools.partial(
        pl.kernel,
        out_type=jax.ShapeDtypeStruct((2, _ACC, _HH), jnp.float32),
        mesh=mesh,
        scratch_types=[
            pltpu.VMEM((_SLAB, 128), jnp.int32),
            pltpu.VMEM((_SLAB, 128), jnp.int32),
            pltpu.VMEM((128, _HH), jnp.float32),
            pltpu.VMEM((128, _HH), jnp.float32),
            pltpu.VMEM_SHARED((_ACC, _HH), jnp.float32),
            pltpu.SemaphoreType.DMA,
            pltpu.SemaphoreType.DMA,
        ],
    )
    def _sc_scatter(table, src2, dst2, init, out,
                    src_v, dst_v, rows_a, rows_b, acc, sem_a, sem_b):
        c = lax.axis_index("c")
        s = lax.axis_index("s")
        pltpu.sync_copy(init.at[c, pl.ds(s * _ACC_T, _ACC_T)],
                        acc.at[pl.ds(s * _ACC_T, _ACC_T)])
        plsc.subcore_barrier()

        def gather(j, buf, sem):
            pltpu.async_copy(table.at[src_v.at[j]], buf, sem)

        def drain_scatter(j, buf, sem):
            pltpu.make_async_copy(table.at[src_v.at[j]], buf, sem).wait()
            pltpu.sync_copy(buf, acc.at[dst_v.at[j]], add=True)

        def outer(t, carry):
            base = s * _ROWS_T + t * _SLAB
            pltpu.sync_copy(src2.at[c, pl.ds(base, _SLAB)], src_v)
            pltpu.sync_copy(dst2.at[pl.ds(base, _SLAB)], dst_v)
            gather(0, rows_a, sem_a)
            gather(1, rows_b, sem_b)

            def pair(k, carry2):
                ja = 2 * k
                drain_scatter(ja, rows_a, sem_a)
                gather(ja + 2, rows_a, sem_a)
                drain_scatter(ja + 1, rows_b, sem_b)
                gather(ja + 3, rows_b, sem_b)
                return carry2

            lax.fori_loop(0, _SLAB // 2 - 1, pair, 0)
            drain_scatter(_SLAB - 2, rows_a, sem_a)
            drain_scatter(_SLAB - 1, rows_b, sem_b)
            return carry

        lax.fori_loop(0, _ROWS_T // _SLAB, outer, 0)
        plsc.subcore_barrier()
        pltpu.sync_copy(acc.at[pl.ds(s * _ACC_T, _ACC_T)],
                        out.at[c, pl.ds(s * _ACC_T, _ACC_T)])

    @functools.partial(
        pl.kernel,
        out_type=jax.ShapeDtypeStruct((2, _ACC, _HH), jnp.float32),
        mesh=mesh,
        scratch_types=[
            pltpu.VMEM((_SLAB, 128), jnp.int32),
            pltpu.VMEM((128, _HH), jnp.float32),
            pltpu.VMEM((128, _HH), jnp.float32),
            pltpu.VMEM_SHARED((_ACC, _HH), jnp.float32),
            pltpu.SemaphoreType.DMA,
            pltpu.SemaphoreType.DMA,
            pltpu.SemaphoreType.DMA,
            pltpu.SemaphoreType.DMA,
        ],
    )
    def _sc_escatter(e2, dste, init, out, dst_v, rows_a, rows_b,
                     acc, sem_a, sem_b, sem_sa, sem_sb):
        c = lax.axis_index("c")
        s = lax.axis_index("s")
        pltpu.sync_copy(init.at[c, pl.ds(s * _ACC_T, _ACC_T)],
                        acc.at[pl.ds(s * _ACC_T, _ACC_T)])
        plsc.subcore_barrier()

        def e_off(k):
            main = (s * _ECH_T + k) * 128
            extra = (16 * _ECH_T + s) * 128
            is_extra = jnp.logical_and(k == _ECH_T, s < _ECH - 16 * _ECH_T)
            return jnp.where(k < _ECH_T, main, jnp.where(is_extra, extra, 0))

        def load(k, buf, sem):
            pltpu.async_copy(e2.at[c, pl.ds(e_off(k), 128)], buf, sem)

        def drain_scatter(t, j, buf, sem, sem_s):
            k = t * _SLAB + j
            pltpu.make_async_copy(e2.at[c, pl.ds(e_off(k), 128)],
                                  buf, sem).wait()
            pltpu.async_copy(buf, acc.at[dst_v.at[j]], sem_s, add=True)

        def wait_scatter(j, buf, sem_s):
            pltpu.make_async_copy(buf, acc.at[dst_v.at[j]], sem_s).wait()

        def outer(t, carry):
            base = s * _ROWS_T + t * _SLAB
            pltpu.sync_copy(dste.at[pl.ds(base, _SLAB)], dst_v)
            load(t * _SLAB, rows_a, sem_a)
            load(t * _SLAB + 1, rows_b, sem_b)

            def pair(k, carry2):
                ja = 2 * k
                drain_scatter(t, ja, rows_a, sem_a, sem_sa)
                drain_scatter(t, ja + 1, rows_b, sem_b, sem_sb)
                wait_scatter(ja, rows_a, sem_sa)
                load(t * _SLAB + ja + 2, rows_a, sem_a)
                wait_scatter(ja + 1, rows_b, sem_sb)
                load(t * _SLAB + ja + 3, rows_b, sem_b)
                return carry2

            lax.fori_loop(0, _SLAB // 2 - 1, pair, 0)
            drain_scatter(t, _SLAB - 2, rows_a, sem_a, sem_sa)
            drain_scatter(t, _SLAB - 1, rows_b, sem_b, sem_sb)
            wait_scatter(_SLAB - 2, rows_a, sem_sa)
            wait_scatter(_SLAB - 1, rows_b, sem_sb)
            return carry

        lax.fori_loop(0, _ROWS_T // _SLAB, outer, 0)
        plsc.subcore_barrier()
        pltpu.sync_copy(acc.at[pl.ds(s * _ACC_T, _ACC_T)],
                        out.at[c, pl.ds(s * _ACC_T, _ACC_T)])

    return _sc_scatter, _sc_escatter


def _proj_body(x_ref, w_ref, out_ref):
    h = jnp.dot(x_ref[...], w_ref[...], preferred_element_type=jnp.float32)
    h = jnp.maximum(h, 0.0)
    out_ref[0, :, :] = h[:, :_HH]
    out_ref[1, :, :] = h[:, _HH:]


def _atom_proj(x, W_i):
    return pl.pallas_call(
        _proj_body,
        grid=(10,),
        in_specs=[pl.BlockSpec((1000, _DF), lambda i: (i, 0)),
                  pl.BlockSpec((_DF, _H), lambda i: (0, 0))],
        out_specs=pl.BlockSpec((2, 1000, _HH), lambda i: (0, i, 0)),
        out_shape=jax.ShapeDtypeStruct((2, _N, _HH), jnp.float32),
    )(x, W_i)


def _edge_proj(ea, W_e):
    return pl.pallas_call(
        _proj_body,
        grid=(80,),
        in_specs=[pl.BlockSpec((4000, _DE), lambda i: (i, 0)),
                  pl.BlockSpec((_DE, _H), lambda i: (0, 0))],
        out_specs=pl.BlockSpec((2, 4000, _HH), lambda i: (0, i, 0)),
        out_shape=jax.ShapeDtypeStruct((2, _E, _HH), jnp.float32),
    )(ea, W_e)


def _update_body(agg_ref, h0_ref, w_ref, out_ref):
    agg = jnp.concatenate([agg_ref[0], agg_ref[1]], axis=1)
    h0 = jnp.concatenate([h0_ref[0], h0_ref[1]], axis=1)
    h = jnp.maximum(
        h0 + jnp.dot(agg, w_ref[...], preferred_element_type=jnp.float32), 0.0)
    out_ref[0, :, :] = h[:, :_HH]
    out_ref[1, :, :] = h[:, _HH:]


def _update(agg2, h0_2, W_h):
    return pl.pallas_call(
        _update_body,
        grid=(10,),
        in_specs=[pl.BlockSpec((2, 1000, _HH), lambda i: (0, i, 0)),
                  pl.BlockSpec((2, 1000, _HH), lambda i: (0, i, 0)),
                  pl.BlockSpec((_H, _H), lambda i: (0, 0))],
        out_specs=pl.BlockSpec((2, 1000, _HH), lambda i: (0, i, 0)),
        out_shape=jax.ShapeDtypeStruct((2, _N, _HH), jnp.float32),
    )(agg2, h0_2, W_h)


def _readout_body(h_ref, mol_ref, w1_ref, b1_ref, w2_ref, b2_ref, out_ref):
    h = jnp.concatenate([h_ref[0], h_ref[1]], axis=1)
    mids = mol_ref[...]
    seg = lax.broadcasted_iota(jnp.int32, (_B, 1), 0)
    mask = (mids == seg).astype(jnp.float32)
    sums = jnp.dot(mask, h, preferred_element_type=jnp.float32)
    counts = jnp.sum(mask, axis=1, keepdims=True)
    mol_vec = sums / jnp.maximum(counts, 1.0)
    hid = jnp.maximum(
        jnp.dot(mol_vec, w1_ref[...], preferred_element_type=jnp.float32)
        + b1_ref[...], 0.0)
    out_ref[...] = (jnp.dot(hid, w2_ref[...], preferred_element_type=jnp.float32)
                    + b2_ref[...])


def _readout(h2, mol2, W1, b1, W2, b2):
    return pl.pallas_call(
        _readout_body,
        out_shape=jax.ShapeDtypeStruct((_B, _TASKS), jnp.float32),
    )(h2, mol2, W1, b1, W2, b2)


def kernel(x, edge_index, edge_attr, mol_ids, W_i, W_e, W_h, W1, b1, W2, b2):
    src = edge_index[0]
    dst = edge_index[1]
    pad = _EPAD - _E
    src_p = jnp.concatenate([src, jnp.zeros((pad,), jnp.int32)])
    dst_p = jnp.concatenate([dst, jnp.full((pad,), _N, jnp.int32)])
    dst2 = dst_p.reshape(_IDX_ROWS, 128)
    src2 = jnp.stack([src_p, src_p + _N]).reshape(2, _IDX_ROWS, 128)
    zinit = jnp.zeros((2, _ACC, _HH), jnp.float32)
    mol2 = mol_ids.reshape(1, _N)

    dmain = dst.reshape(_ECH, 128)
    n_left = _ECH - 16 * _ECH_T
    ex_first = jnp.concatenate(
        [dmain[16 * _ECH_T:],
         jnp.full((16 - n_left, 128), _N, jnp.int32)]).reshape(16, 1, 128)
    ex_rest = jnp.full((16, _ROWS_T - _ECH_T - 1, 128), _N, jnp.int32)
    dste = jnp.concatenate(
        [dmain[:16 * _ECH_T].reshape(16, _ECH_T, 128), ex_first, ex_rest],
        axis=1).reshape(_IDX_ROWS, 128)

    sc_scatter, sc_escatter = _sc_fns()
    h0_2 = _atom_proj(x, W_i)
    e2 = _edge_proj(edge_attr, W_e)
    e_agg = sc_escatter(e2, dste, zinit)
    h2 = h0_2
    for _ in range(_DEPTH):
        agg2 = sc_scatter(h2.reshape(2 * _N, _HH), src2, dst2, e_agg)
        h2 = _update(agg2, h0_2, W_h)
    return _readout(h2, mol2, W1, b1, W2, b2)


# speedup vs baseline: 1.0314x; 1.0314x over previous
import functools

import jax
import jax.numpy as jnp
from jax import lax
from jax.experimental import pallas as pl
from jax.experimental.pallas import tpu as pltpu
from jax.experimental.pallas import tpu_sc as plsc

_N = 10000
_E = 320000
_DF = 128
_DE = 16
_H = 256
_HH = 128
_B = 256
_TASKS = 12
_DEPTH = 3

_EPAD = 327680
_IDX_ROWS = _EPAD // 128
_ACC = 10240
_NSUB = 16
_ROWS_T = _IDX_ROWS // _NSUB
_ACC_T = _ACC // _NSUB
_SLAB = 32

_ECH = _E // 128
_ECH_T = 156


@functools.cache
def _sc_fns():
    mesh = plsc.VectorSubcoreMesh(core_axis_name="c", subcore_axis_name="s")

    @functools.partial(
        pl.kernel,
        out_type=jax.ShapeDtypeStruct((2, _ACC, _HH), jnp.float32),
        mesh=mesh,
        scratch_types=[
            pltpu.VMEM((_SLAB, 128), jnp.int32),
            pltpu.VMEM((_SLAB, 128), jnp.int32),
            pltpu.VMEM((128, _HH), jnp.float32),
            pltpu.VMEM((128, _HH), jnp.float32),
            pltpu.VMEM_SHARED((_ACC, _HH), jnp.float32),
            pltpu.SemaphoreType.DMA,
            pltpu.SemaphoreType.DMA,
        ],
    )
    def _sc_scatter(table, src2, dst2, init, out,
                    src_v, dst_v, rows_a, rows_b, acc, sem_a, sem_b):
        c = lax.axis_index("c")
        s = lax.axis_index("s")
        pltpu.sync_copy(init.at[c, pl.ds(s * _ACC_T, _ACC_T)],
                        acc.at[pl.ds(s * _ACC_T, _ACC_T)])
        plsc.subcore_barrier()

        def gather(j, buf, sem):
            pltpu.async_copy(table.at[src_v.at[j]], buf, sem)

        def drain_scatter(j, buf, sem):
            pltpu.make_async_copy(table.at[src_v.at[j]], buf, sem).wait()
            pltpu.sync_copy(buf, acc.at[dst_v.at[j]], add=True)

        def outer(t, carry):
            base = s * _ROWS_T + t * _SLAB
            pltpu.sync_copy(src2.at[c, pl.ds(base, _SLAB)], src_v)
            pltpu.sync_copy(dst2.at[pl.ds(base, _SLAB)], dst_v)
            gather(0, rows_a, sem_a)
            gather(1, rows_b, sem_b)

            def pair(k, carry2):
                ja = 2 * k
                drain_scatter(ja, rows_a, sem_a)
                gather(ja + 2, rows_a, sem_a)
                drain_scatter(ja + 1, rows_b, sem_b)
                gather(ja + 3, rows_b, sem_b)
                return carry2

            lax.fori_loop(0, _SLAB // 2 - 1, pair, 0)
            drain_scatter(_SLAB - 2, rows_a, sem_a)
            drain_scatter(_SLAB - 1, rows_b, sem_b)
            return carry

        lax.fori_loop(0, _ROWS_T // _SLAB, outer, 0)
        plsc.subcore_barrier()
        pltpu.sync_copy(acc.at[pl.ds(s * _ACC_T, _ACC_T)],
                        out.at[c, pl.ds(s * _ACC_T, _ACC_T)])

    @functools.partial(
        pl.kernel,
        out_type=jax.ShapeDtypeStruct((2, _ACC, _HH), jnp.float32),
        mesh=mesh,
        scratch_types=[
            pltpu.VMEM((_SLAB, 128), jnp.int32),
            pltpu.VMEM((128, _HH), jnp.float32),
            pltpu.VMEM((128, _HH), jnp.float32),
            pltpu.VMEM_SHARED((_ACC, _HH), jnp.float32),
            pltpu.SemaphoreType.DMA,
            pltpu.SemaphoreType.DMA,
        ],
    )
    def _sc_escatter(e2, dste, init, out, dst_v, rows_a, rows_b,
                     acc, sem_a, sem_b):
        c = lax.axis_index("c")
        s = lax.axis_index("s")
        pltpu.sync_copy(init.at[c, pl.ds(s * _ACC_T, _ACC_T)],
                        acc.at[pl.ds(s * _ACC_T, _ACC_T)])
        plsc.subcore_barrier()

        def e_off(k):
            main = (s * _ECH_T + k) * 128
            extra = (16 * _ECH_T + s) * 128
            is_extra = jnp.logical_and(k == _ECH_T, s < _ECH - 16 * _ECH_T)
            return jnp.where(k < _ECH_T, main, jnp.where(is_extra, extra, 0))

        def load(k, buf, sem):
            pltpu.async_copy(e2.at[c, pl.ds(e_off(k), 128)], buf, sem)

        def drain_scatter(t, j, buf, sem):
            k = t * _SLAB + j
            pltpu.make_async_copy(e2.at[c, pl.ds(e_off(k), 128)],
                                  buf, sem).wait()
            pltpu.sync_copy(buf, acc.at[dst_v.at[j]], add=True)

        def outer(t, carry):
            base = s * _ROWS_T + t * _SLAB
            pltpu.sync_copy(dste.at[pl.ds(base, _SLAB)], dst_v)
            load(t * _SLAB, rows_a, sem_a)
            load(t * _SLAB + 1, rows_b, sem_b)

            def pair(k, carry2):
                ja = 2 * k
                drain_scatter(t, ja, rows_a, sem_a)
                load(t * _SLAB + ja + 2, rows_a, sem_a)
                drain_scatter(t, ja + 1, rows_b, sem_b)
                load(t * _SLAB + ja + 3, rows_b, sem_b)
                return carry2

            lax.fori_loop(0, _SLAB // 2 - 1, pair, 0)
            drain_scatter(t, _SLAB - 2, rows_a, sem_a)
            drain_scatter(t, _SLAB - 1, rows_b, sem_b)
            return carry

        lax.fori_loop(0, _ROWS_T // _SLAB, outer, 0)
        plsc.subcore_barrier()
        pltpu.sync_copy(acc.at[pl.ds(s * _ACC_T, _ACC_T)],
                        out.at[c, pl.ds(s * _ACC_T, _ACC_T)])

    return _sc_scatter, _sc_escatter


def _proj_body(x_ref, w_ref, out_ref):
    h = jnp.dot(x_ref[...], w_ref[...], preferred_element_type=jnp.float32)
    h = jnp.maximum(h, 0.0)
    out_ref[0, :, :] = h[:, :_HH]
    out_ref[1, :, :] = h[:, _HH:]


def _atom_proj(x, W_i):
    return pl.pallas_call(
        _proj_body,
        grid=(10,),
        in_specs=[pl.BlockSpec((1000, _DF), lambda i: (i, 0)),
                  pl.BlockSpec((_DF, _H), lambda i: (0, 0))],
        out_specs=pl.BlockSpec((2, 1000, _HH), lambda i: (0, i, 0)),
        out_shape=jax.ShapeDtypeStruct((2, _N, _HH), jnp.float32),
    )(x, W_i)


def _edge_proj(ea, W_e):
    return pl.pallas_call(
        _proj_body,
        grid=(80,),
        in_specs=[pl.BlockSpec((4000, _DE), lambda i: (i, 0)),
                  pl.BlockSpec((_DE, _H), lambda i: (0, 0))],
        out_specs=pl.BlockSpec((2, 4000, _HH), lambda i: (0, i, 0)),
        out_shape=jax.ShapeDtypeStruct((2, _E, _HH), jnp.float32),
    )(ea, W_e)


def _update_body(agg_ref, h0_ref, w_ref, out_ref):
    agg = jnp.concatenate([agg_ref[0], agg_ref[1]], axis=1)
    h0 = jnp.concatenate([h0_ref[0], h0_ref[1]], axis=1)
    h = jnp.maximum(
        h0 + jnp.dot(agg, w_ref[...], preferred_element_type=jnp.float32), 0.0)
    out_ref[0, :, :] = h[:, :_HH]
    out_ref[1, :, :] = h[:, _HH:]


def _update(agg2, h0_2, W_h):
    return pl.pallas_call(
        _update_body,
        grid=(10,),
        in_specs=[pl.BlockSpec((2, 1000, _HH), lambda i: (0, i, 0)),
                  pl.BlockSpec((2, 1000, _HH), lambda i: (0, i, 0)),
                  pl.BlockSpec((_H, _H), lambda i: (0, 0))],
        out_specs=pl.BlockSpec((2, 1000, _HH), lambda i: (0, i, 0)),
        out_shape=jax.ShapeDtypeStruct((2, _N, _HH), jnp.float32),
    )(agg2, h0_2, W_h)


def _readout_body(h_ref, mol_ref, w1_ref, b1_ref, w2_ref, b2_ref, out_ref):
    h = jnp.concatenate([h_ref[0], h_ref[1]], axis=1)
    mids = mol_ref[...]
    seg = lax.broadcasted_iota(jnp.int32, (_B, 1), 0)
    mask = (mids == seg).astype(jnp.float32)
    sums = jnp.dot(mask, h, preferred_element_type=jnp.float32)
    counts = jnp.sum(mask, axis=1, keepdims=True)
    mol_vec = sums / jnp.maximum(counts, 1.0)
    hid = jnp.maximum(
        jnp.dot(mol_vec, w1_ref[...], preferred_element_type=jnp.float32)
        + b1_ref[...], 0.0)
    out_ref[...] = (jnp.dot(hid, w2_ref[...], preferred_element_type=jnp.float32)
                    + b2_ref[...])


def _readout(h2, mol2, W1, b1, W2, b2):
    return pl.pallas_call(
        _readout_body,
        out_shape=jax.ShapeDtypeStruct((_B, _TASKS), jnp.float32),
    )(h2, mol2, W1, b1, W2, b2)


def kernel(x, edge_index, edge_attr, mol_ids, W_i, W_e, W_h, W1, b1, W2, b2):
    src = edge_index[0]
    dst = edge_index[1]
    pad = _EPAD - _E
    src_p = jnp.concatenate([src, jnp.zeros((pad,), jnp.int32)])
    dst_p = jnp.concatenate([dst, jnp.full((pad,), _N, jnp.int32)])
    dst2 = dst_p.reshape(_IDX_ROWS, 128)
    src2 = jnp.stack([src_p, src_p + _N]).reshape(2, _IDX_ROWS, 128)
    zinit = jnp.zeros((2, _ACC, _HH), jnp.float32)
    mol2 = mol_ids.reshape(1, _N)

    dmain = dst.reshape(_ECH, 128)
    n_left = _ECH - 16 * _ECH_T
    ex_first = jnp.concatenate(
        [dmain[16 * _ECH_T:],
         jnp.full((16 - n_left, 128), _N, jnp.int32)]).reshape(16, 1, 128)
    ex_rest = jnp.full((16, _ROWS_T - _ECH_T - 1, 128), _N, jnp.int32)
    dste = jnp.concatenate(
        [dmain[:16 * _ECH_T].reshape(16, _ECH_T, 128), ex_first, ex_rest],
        axis=1).reshape(_IDX_ROWS, 128)

    sc_scatter, sc_escatter = _sc_fns()
    h0_2 = _atom_proj(x, W_i)
    e2 = _edge_proj(edge_attr, W_e)
    e_agg = sc_escatter(e2, dste, zinit)
    h2 = h0_2
    for _ in range(_DEPTH):
        agg2 = sc_scatter(h2.reshape(2 * _N, _HH), src2, dst2, e_agg)
        h2 = _update(agg2, h0_2, W_h)
    return _readout(h2, mol2, W1, b1, W2, b2)
